# Initial kernel scaffold; baseline (speedup 1.0000x reference)
import jax
import jax.numpy as jnp
from jax.experimental import pallas as pl


def kernel(features, W_inter, b_inter, W_cls, b_cls, W_reg, b_reg):
    raise NotImplementedError("write your pallas kernel here")


# speedup vs baseline: 1.3010x; 1.3010x over previous
import jax
import jax.numpy as jnp
from jax.experimental import pallas as pl
from jax.experimental.pallas import tpu as pltpu

B, C, H, W = 4, 256, 128, 128
O = 256
NCLS, NREG = 9, 36
TH = 32
HB = H // TH
WP = 136


def _rpn_kernel(x_ref, wtap_ref, bi_ref, wc_ref, bc_ref, wr_ref, br_ref,
                cls_ref, reg_ref):
    x = x_ref[0, 0]
    acc = jnp.zeros((TH, W, O), jnp.float32)
    for kx in range(3):
        xs = x[:, kx:kx + W, :]
        for ky in range(3):
            a = xs[ky:ky + TH]
            wk = wtap_ref[ky, kx]
            acc = acc + jax.lax.dot_general(
                a, wk, (((2,), (0,)), ((), ())),
                preferred_element_type=jnp.float32)
    inter = jnp.maximum(acc + bi_ref[0, 0], 0.0)
    ib = inter.astype(jnp.bfloat16)
    cls = jax.lax.dot_general(ib, wc_ref[...], (((2,), (0,)), ((), ())),
                              preferred_element_type=jnp.float32)
    reg = jax.lax.dot_general(ib, wr_ref[...], (((2,), (0,)), ((), ())),
                              preferred_element_type=jnp.float32)
    cls_ref[0] = jax.nn.sigmoid(cls + bc_ref[0])
    reg_ref[0] = reg + br_ref[0]


def kernel(features, W_inter, b_inter, W_cls, b_cls, W_reg, b_reg):
    x = jnp.transpose(features, (0, 2, 3, 1)).astype(jnp.bfloat16)
    xp = jnp.pad(x, ((0, 0), (1, 1), (1, WP - W - 1), (0, 0)))
    xw = jnp.stack([xp[:, i * TH:i * TH + TH + 2] for i in range(HB)], axis=1)

    wtap = jnp.transpose(W_inter, (2, 3, 1, 0)).astype(jnp.bfloat16)
    wc = jnp.transpose(W_cls[:, :, 0, 0], (1, 0)).astype(jnp.bfloat16)
    wr = jnp.transpose(W_reg[:, :, 0, 0], (1, 0)).astype(jnp.bfloat16)
    bi = b_inter.reshape(1, 1, O)
    bc = b_cls.reshape(1, 1, NCLS)
    br = b_reg.reshape(1, 1, NREG)

    grid = (B, HB)
    cls_nhwc, reg_nhwc = pl.pallas_call(
        _rpn_kernel,
        grid=grid,
        in_specs=[
            pl.BlockSpec((1, 1, TH + 2, WP, C), lambda b, h: (b, h, 0, 0, 0)),
            pl.BlockSpec((3, 3, C, O), lambda b, h: (0, 0, 0, 0)),
            pl.BlockSpec((1, 1, O), lambda b, h: (0, 0, 0)),
            pl.BlockSpec((C, NCLS), lambda b, h: (0, 0)),
            pl.BlockSpec((1, 1, NCLS), lambda b, h: (0, 0, 0)),
            pl.BlockSpec((C, NREG), lambda b, h: (0, 0)),
            pl.BlockSpec((1, 1, NREG), lambda b, h: (0, 0, 0)),
        ],
        out_specs=[
            pl.BlockSpec((1, TH, W, NCLS), lambda b, h: (b, h, 0, 0)),
            pl.BlockSpec((1, TH, W, NREG), lambda b, h: (b, h, 0, 0)),
        ],
        out_shape=[
            jax.ShapeDtypeStruct((B, H, W, NCLS), jnp.float32),
            jax.ShapeDtypeStruct((B, H, W, NREG), jnp.float32),
        ],
    )(xw, wtap, bi, wc, bc, wr, br)

    cls_out = jnp.transpose(cls_nhwc, (0, 3, 1, 2))
    reg_out = jnp.transpose(reg_nhwc, (0, 3, 1, 2))
    return (cls_out, reg_out)


# speedup vs baseline: 1.4708x; 1.1305x over previous
import jax
import jax.numpy as jnp
from jax import lax
from jax.experimental import pallas as pl
from jax.experimental.pallas import tpu as pltpu

B, C, H, W = 4, 256, 128, 128
O = 256
NCLS, NREG = 9, 36
TH = 32
HB = H // TH
NSTEPS = B * HB
WP = 136


def _start_fetch(x_hbm, buf, sem, step, slot):
    b = step // HB
    h = step % HB
    pltpu.make_async_copy(
        x_hbm.at[b, pl.ds(h * TH, TH + 2)], buf.at[slot], sem.at[slot],
    ).start()


def _rpn_kernel(x_hbm, wtap_ref, bi_ref, wc_ref, bc_ref, wr_ref, br_ref,
                cls_ref, reg_ref, buf, sem):
    b = pl.program_id(0)
    h = pl.program_id(1)
    step = b * HB + h
    slot = lax.rem(step, 2)

    @pl.when(step == 0)
    def _():
        _start_fetch(x_hbm, buf, sem, 0, 0)

    @pl.when(step + 1 < NSTEPS)
    def _():
        _start_fetch(x_hbm, buf, sem, step + 1, lax.rem(step + 1, 2))

    pltpu.make_async_copy(
        x_hbm.at[0, pl.ds(0, TH + 2)], buf.at[slot], sem.at[slot],
    ).wait()

    x = buf[slot]
    cat = jnp.concatenate(
        [x[ky:ky + TH, kx:kx + W, :] for ky in range(3) for kx in range(3)],
        axis=2)
    acc = jax.lax.dot_general(cat, wtap_ref[...], (((2,), (0,)), ((), ())),
                              preferred_element_type=jnp.float32)
    inter = jnp.maximum(acc + bi_ref[0, 0], 0.0)
    ib = inter.astype(jnp.bfloat16)
    cls = jax.lax.dot_general(ib, wc_ref[...], (((2,), (0,)), ((), ())),
                              preferred_element_type=jnp.float32)
    reg = jax.lax.dot_general(ib, wr_ref[...], (((2,), (0,)), ((), ())),
                              preferred_element_type=jnp.float32)
    cls_ref[0] = jax.nn.sigmoid(cls + bc_ref[0])
    reg_ref[0] = reg + br_ref[0]


def kernel(features, W_inter, b_inter, W_cls, b_cls, W_reg, b_reg):
    x = jnp.transpose(features, (0, 2, 3, 1)).astype(jnp.bfloat16)
    xp = jnp.pad(x, ((0, 0), (1, 1), (1, WP - W - 1), (0, 0)))

    wtap = jnp.transpose(W_inter, (2, 3, 1, 0)).reshape(9 * C, O).astype(jnp.bfloat16)
    wc = jnp.transpose(W_cls[:, :, 0, 0], (1, 0)).astype(jnp.bfloat16)
    wr = jnp.transpose(W_reg[:, :, 0, 0], (1, 0)).astype(jnp.bfloat16)
    bi = b_inter.reshape(1, 1, O)
    bc = b_cls.reshape(1, 1, NCLS)
    br = b_reg.reshape(1, 1, NREG)

    grid = (B, HB)
    cls_nhwc, reg_nhwc = pl.pallas_call(
        _rpn_kernel,
        grid=grid,
        in_specs=[
            pl.BlockSpec(memory_space=pl.ANY),
            pl.BlockSpec((9 * C, O), lambda b, h: (0, 0)),
            pl.BlockSpec((1, 1, O), lambda b, h: (0, 0, 0)),
            pl.BlockSpec((C, NCLS), lambda b, h: (0, 0)),
            pl.BlockSpec((1, 1, NCLS), lambda b, h: (0, 0, 0)),
            pl.BlockSpec((C, NREG), lambda b, h: (0, 0)),
            pl.BlockSpec((1, 1, NREG), lambda b, h: (0, 0, 0)),
        ],
        out_specs=[
            pl.BlockSpec((1, TH, W, NCLS), lambda b, h: (b, h, 0, 0)),
            pl.BlockSpec((1, TH, W, NREG), lambda b, h: (b, h, 0, 0)),
        ],
        out_shape=[
            jax.ShapeDtypeStruct((B, H, W, NCLS), jnp.float32),
            jax.ShapeDtypeStruct((B, H, W, NREG), jnp.float32),
        ],
        scratch_shapes=[
            pltpu.VMEM((2, TH + 2, WP, C), jnp.bfloat16),
            pltpu.SemaphoreType.DMA((2,)),
        ],
    )(xp, wtap, bi, wc, bc, wr, br)

    cls_out = jnp.transpose(cls_nhwc, (0, 3, 1, 2))
    reg_out = jnp.transpose(reg_nhwc, (0, 3, 1, 2))
    return (cls_out, reg_out)


# speedup vs baseline: 2.5363x; 1.7244x over previous
import jax
import jax.numpy as jnp
from jax import lax
from jax.experimental import pallas as pl
from jax.experimental.pallas import tpu as pltpu

B, C, H, W = 4, 256, 128, 128
O = 256
NCLS, NREG = 9, 36
TH = 32
HB = H // TH
NSTEPS = B * HB
WP = 136
RS = TH + 2


def _start_fetch(x_hbm, buf, sem, step, slot):
    b = step // HB
    h = step % HB

    @pl.when(h == 0)
    def _():
        pltpu.make_async_copy(
            x_hbm.at[b, :, pl.ds(0, TH + 1), :],
            buf.at[slot, :, pl.ds(1, TH + 1), :], sem.at[slot]).start()

    @pl.when(jnp.logical_and(h > 0, h < HB - 1))
    def _():
        pltpu.make_async_copy(
            x_hbm.at[b, :, pl.ds(h * TH - 1, TH + 2), :],
            buf.at[slot, :, pl.ds(0, TH + 2), :], sem.at[slot]).start()

    @pl.when(h == HB - 1)
    def _():
        pltpu.make_async_copy(
            x_hbm.at[b, :, pl.ds(H - TH - 1, TH + 1), :],
            buf.at[slot, :, pl.ds(0, TH + 1), :], sem.at[slot]).start()


def _wait_fetch(x_hbm, buf, sem, step, slot):
    h = step % HB

    @pl.when(jnp.logical_or(h == 0, h == HB - 1))
    def _():
        pltpu.make_async_copy(
            x_hbm.at[0, :, pl.ds(0, TH + 1), :],
            buf.at[slot, :, pl.ds(0, TH + 1), :], sem.at[slot]).wait()

    @pl.when(jnp.logical_and(h > 0, h < HB - 1))
    def _():
        pltpu.make_async_copy(
            x_hbm.at[0, :, pl.ds(0, TH + 2), :],
            buf.at[slot, :, pl.ds(0, TH + 2), :], sem.at[slot]).wait()


def _rpn_kernel(x_hbm, wtap_ref, bi_ref, wc_ref, bc_ref, wr_ref, br_ref,
                cls_ref, reg_ref, buf, xtp, sem):
    b = pl.program_id(0)
    h = pl.program_id(1)
    step = b * HB + h
    slot = lax.rem(step, 2)

    @pl.when(step == 0)
    def _():
        _start_fetch(x_hbm, buf, sem, 0, 0)
        xtp[...] = jnp.zeros((RS, WP, C), jnp.bfloat16)

    @pl.when(step + 1 < NSTEPS)
    def _():
        _start_fetch(x_hbm, buf, sem, step + 1, lax.rem(step + 1, 2))

    _wait_fetch(x_hbm, buf, sem, step, slot)

    @pl.when(h == 0)
    def _():
        buf[slot, :, 0:1, :] = jnp.zeros((C, 1, W), jnp.float32)

    @pl.when(h == HB - 1)
    def _():
        buf[slot, :, TH + 1:TH + 2, :] = jnp.zeros((C, 1, W), jnp.float32)

    xt = jnp.transpose(buf[slot].reshape(C, RS * W), (1, 0))
    xtp[:, 1:W + 1, :] = xt.astype(jnp.bfloat16).reshape(RS, W, C)

    x = xtp[...]
    cat = jnp.concatenate(
        [x[ky:ky + TH, kx:kx + W, :] for ky in range(3) for kx in range(3)],
        axis=2)
    acc = jax.lax.dot_general(cat, wtap_ref[...], (((2,), (0,)), ((), ())),
                              preferred_element_type=jnp.float32)
    inter = jnp.maximum(acc + bi_ref[0, 0], 0.0)
    ib = inter.astype(jnp.bfloat16)
    cls = jax.lax.dot_general(ib, wc_ref[...], (((2,), (0,)), ((), ())),
                              preferred_element_type=jnp.float32)
    reg = jax.lax.dot_general(ib, wr_ref[...], (((2,), (0,)), ((), ())),
                              preferred_element_type=jnp.float32)
    cls = jax.nn.sigmoid(cls + bc_ref[0, 0])
    reg = reg + br_ref[0, 0]
    cls_ref[0] = jnp.transpose(cls.reshape(TH * W, NCLS), (1, 0)).reshape(
        NCLS, TH, W)
    reg_ref[0] = jnp.transpose(reg.reshape(TH * W, NREG), (1, 0)).reshape(
        NREG, TH, W)


def kernel(features, W_inter, b_inter, W_cls, b_cls, W_reg, b_reg):
    wtap = jnp.transpose(W_inter, (2, 3, 1, 0)).reshape(9 * C, O).astype(jnp.bfloat16)
    wc = jnp.transpose(W_cls[:, :, 0, 0], (1, 0)).astype(jnp.bfloat16)
    wr = jnp.transpose(W_reg[:, :, 0, 0], (1, 0)).astype(jnp.bfloat16)
    bi = b_inter.reshape(1, 1, O)
    bc = b_cls.reshape(1, 1, NCLS)
    br = b_reg.reshape(1, 1, NREG)

    grid = (B, HB)
    cls_out, reg_out = pl.pallas_call(
        _rpn_kernel,
        grid=grid,
        in_specs=[
            pl.BlockSpec(memory_space=pl.ANY),
            pl.BlockSpec((9 * C, O), lambda b, h: (0, 0)),
            pl.BlockSpec((1, 1, O), lambda b, h: (0, 0, 0)),
            pl.BlockSpec((C, NCLS), lambda b, h: (0, 0)),
            pl.BlockSpec((1, 1, NCLS), lambda b, h: (0, 0, 0)),
            pl.BlockSpec((C, NREG), lambda b, h: (0, 0)),
            pl.BlockSpec((1, 1, NREG), lambda b, h: (0, 0, 0)),
        ],
        out_specs=[
            pl.BlockSpec((1, NCLS, TH, W), lambda b, h: (b, 0, h, 0)),
            pl.BlockSpec((1, NREG, TH, W), lambda b, h: (b, 0, h, 0)),
        ],
        out_shape=[
            jax.ShapeDtypeStruct((B, NCLS, H, W), jnp.float32),
            jax.ShapeDtypeStruct((B, NREG, H, W), jnp.float32),
        ],
        scratch_shapes=[
            pltpu.VMEM((2, C, RS, W), jnp.float32),
            pltpu.VMEM((RS, WP, C), jnp.bfloat16),
            pltpu.SemaphoreType.DMA((2,)),
        ],
    )(features, wtap, bi, wc, bc, wr, br)

    return (cls_out, reg_out)
